# Initial kernel scaffold; baseline (speedup 1.0000x reference)
import jax
import jax.numpy as jnp
from jax.experimental import pallas as pl


def kernel(node_emb, edge_index, edge_type, W, b_lin, rel_emb_gat, a_src, a_dst, a_rel, bias_out, rel_emb_scorer, src_ids, rel_ids, dst_ids):
    raise NotImplementedError("write your pallas kernel here")


# speedup vs baseline: 40.3396x; 40.3396x over previous
import functools

import jax
import jax.numpy as jnp
from jax import lax
from jax.experimental import pallas as pl
from jax.experimental.pallas import tpu as pltpu
from jax.experimental.pallas import tpu_sc as plsc

F32 = jnp.float32
I32 = jnp.int32

_N = 10000
_E = 320000
_DIN = 128
_H = 2
_D = 64
_HD = _H * _D
_R = 40
_B = 16384

_NC = 2
_NS = 16
_NW = _NC * _NS
_L = 16

_EC = 80
_EPW = _E // _NW
_ECH = _EPW // _EC
_NP = 10240
_NPT = _NP // _NS
_ZR = 128

_BC = 128
_BPW = _B // _NW
_BCH = _BPW // _BC

_AUG = 144

_mesh = plsc.VectorSubcoreMesh(core_axis_name="c", subcore_axis_name="s")


def _prep_body(ne_ref, w_ref, b_ref, asrc_ref, adst_ref, arel_ref, rg_ref,
               haug_ref, adstt_ref, relsc_ref):
    h = jnp.dot(ne_ref[...], w_ref[...], preferred_element_type=F32) + b_ref[...]
    scols = []
    dcols = []
    for hh in range(_H):
        hs = h[:, hh * _D:(hh + 1) * _D]
        scols.append(jnp.sum(hs * asrc_ref[hh:hh + 1, :], axis=1, keepdims=True))
        dcols.append(jnp.sum(hs * adst_ref[hh:hh + 1, :], axis=1, keepdims=True))
    haug_ref[...] = jnp.concatenate(
        [h] + scols + [jnp.zeros((_N, _AUG - _HD - _H), F32)], axis=1)
    adstt_ref[...] = jnp.concatenate(dcols, axis=1)
    rg = rg_ref[...]
    rcols = []
    for hh in range(_H):
        v = arel_ref[hh:hh + 1, :]
        rcols.append(jnp.sum(rg[:, hh * _D:(hh + 1) * _D] * v,
                             axis=1, keepdims=True))
    rcols.append(jnp.zeros((_R, 8 - _H), F32))
    relsc_ref[...] = jnp.concatenate(rcols, axis=1)


_prep = pl.pallas_call(
    _prep_body,
    out_shape=(
        jax.ShapeDtypeStruct((_N, _AUG), F32),
        jax.ShapeDtypeStruct((_N, 2), F32),
        jax.ShapeDtypeStruct((_R, 8), F32),
    ),
)


def _edge_body(h_hbm, srcs, dsts, ets, adst_hbm, relsc_hbm, relrow_hbm,
               agg_out,
               adst_v, relsc_v, relrow_v, src_v, dst_v, et_v,
               hrows_v, agg_sp, sem):
    c = lax.axis_index("c")
    s = lax.axis_index("s")
    wid = s * _NC + c

    pltpu.sync_copy(adst_hbm, adst_v)
    pltpu.sync_copy(relsc_hbm, relsc_v)
    pltpu.sync_copy(relrow_hbm, relrow_v)

    zvec = jnp.zeros((_L,), F32)

    def zrow(i, carry):
        for j in range(_AUG // _L):
            hrows_v[i, pl.ds(j * _L, _L)] = zvec
        return carry

    lax.fori_loop(0, _EC, zrow, 0)

    rbase = s * _NPT
    for k in range(_NPT // _EC):
        pltpu.sync_copy(hrows_v, agg_sp.at[pl.ds(rbase + k * _EC, _EC)])

    plsc.subcore_barrier()

    iota = lax.iota(I32, _L)
    lane0 = iota == 0
    lane1 = iota == 1
    base0 = wid * _EPW

    def chunk_body(ci, carry):
        base = base0 + ci * _EC
        pltpu.sync_copy(srcs.at[pl.ds(base, _EC)], src_v)
        pltpu.sync_copy(dsts.at[pl.ds(base, _EC)], dst_v)
        pltpu.sync_copy(ets.at[pl.ds(base, _EC)], et_v)
        pltpu.async_copy(h_hbm.at[src_v], hrows_v, sem).wait()

        def group(g, gcarry):
            off = g * _L
            dv = dst_v[pl.ds(off, _L)] * 2
            tv = et_v[pl.ds(off, _L)]
            b0 = plsc.load_gather(adst_v, [dv])
            b1 = plsc.load_gather(adst_v, [dv + 1])
            c0 = plsc.load_gather(relsc_v, [tv * 8])
            c1 = plsc.load_gather(relsc_v, [tv * 8 + 1])
            p0 = b0 + c0
            p1 = b1 + c1
            rbv = tv * _HD
            for i in range(_L):
                sa = hrows_v[off + i, pl.ds(_HD, _L)]
                l0 = sa[0] + p0[i]
                l1 = sa[1] + p1[i]
                l0 = jnp.where(l0 >= 0.0, l0, 0.2 * l0)
                l1 = jnp.where(l1 >= 0.0, l1, 0.2 * l1)
                x0 = jnp.exp(jnp.full((_L,), l0, F32))
                x1 = jnp.exp(jnp.full((_L,), l1, F32))
                rb = rbv[i]
                hrows_v[off + i, pl.ds(_HD, _L)] = jnp.where(
                    lane0, x0, jnp.where(lane1, x1, 0.0))
                for j in range(_HD // _L):
                    xh = x0 if j < (_D // _L) else x1
                    hv = hrows_v[off + i, pl.ds(j * _L, _L)]
                    rv = relrow_v[pl.ds(rb + j * _L, _L)]
                    hrows_v[off + i, pl.ds(j * _L, _L)] = xh * (hv + rv)
            return gcarry

        lax.fori_loop(0, _EC // _L, group, 0)
        pltpu.sync_copy(hrows_v, agg_sp.at[dst_v], add=True)
        return carry

    lax.fori_loop(0, _ECH, chunk_body, 0)

    plsc.subcore_barrier()

    pltpu.sync_copy(agg_sp.at[pl.ds(rbase, _NPT)], agg_out.at[c, pl.ds(rbase, _NPT)])


_edge = functools.partial(
    pl.kernel,
    out_type=jax.ShapeDtypeStruct((_NC, _NP, _AUG), F32),
    mesh=_mesh,
    scratch_types=[
        pltpu.VMEM((_N * 2,), F32),
        pltpu.VMEM((_R * 8,), F32),
        pltpu.VMEM((_R * _HD,), F32),
        pltpu.VMEM((_EC,), I32),
        pltpu.VMEM((_EC,), I32),
        pltpu.VMEM((_EC,), I32),
        pltpu.VMEM((_EC, _AUG), F32),
        pltpu.VMEM_SHARED((_NP, _AUG), F32),
        pltpu.SemaphoreType.DMA,
    ],
    compiler_params=pltpu.CompilerParams(needs_layout_passes=False,
                                         use_tc_tiling_on_sc=False),
)(_edge_body)


def _final_body(agg_ref, bias_ref, out_ref):
    a = agg_ref[0, :_N, :_HD] + agg_ref[1, :_N, :_HD]
    dn = (agg_ref[0, :_N, _HD:_HD + _H] + agg_ref[1, :_N, _HD:_HD + _H])
    d0 = jnp.broadcast_to(dn[:, 0:1], (_N, _D))
    d1 = jnp.broadcast_to(dn[:, 1:2], (_N, _D))
    denw = jnp.concatenate([d0, d1], axis=1)
    pre = a / (denw + 1e-16) + bias_ref[...]
    out_ref[...] = jnp.where(pre > 0.0, pre, jnp.exp(pre) - 1.0)


_final = pl.pallas_call(
    _final_body,
    out_shape=jax.ShapeDtypeStruct((_N, _HD), F32),
)


def _score_body(refined_hbm, relsc2_hbm, sids, rids, dids, out_hbm,
                relsc2_v, sid_v, rid_v, did_v, srows_v, drows_v, out_v,
                sem1, sem2):
    c = lax.axis_index("c")
    s = lax.axis_index("s")
    wid = s * _NC + c
    pltpu.sync_copy(relsc2_hbm, relsc2_v)
    base0 = wid * _BPW
    for ci in range(_BCH):
        base = base0 + ci * _BC
        pltpu.sync_copy(sids.at[pl.ds(base, _BC)], sid_v)
        pltpu.sync_copy(rids.at[pl.ds(base, _BC)], rid_v)
        pltpu.sync_copy(dids.at[pl.ds(base, _BC)], did_v)
        cp1 = pltpu.async_copy(refined_hbm.at[sid_v], srows_v, sem1)
        cp2 = pltpu.async_copy(refined_hbm.at[did_v], drows_v, sem2)
        cp1.wait()
        cp2.wait()

        iota = lax.iota(I32, _L)

        def group(g, gcarry):
            off = g * _L
            rv = rid_v[pl.ds(off, _L)] * _HD
            res = jnp.zeros((_L,), F32)
            for i in range(_L):
                rb = rv[i]
                acc = jnp.zeros((_L,), F32)
                for j in range(_HD // _L):
                    acc = acc + (srows_v[off + i, pl.ds(j * _L, _L)]
                                 * drows_v[off + i, pl.ds(j * _L, _L)]
                                 * relsc2_v[pl.ds(rb + j * _L, _L)])
                tot = jnp.sum(acc, axis=0)
                res = jnp.where(iota == i, jnp.full((_L,), tot, F32), res)
            out_v[pl.ds(off, _L)] = res
            return gcarry

        lax.fori_loop(0, _BC // _L, group, 0)
        pltpu.sync_copy(out_v, out_hbm.at[pl.ds(base, _BC)])


_score = functools.partial(
    pl.kernel,
    out_type=jax.ShapeDtypeStruct((_B,), F32),
    mesh=_mesh,
    scratch_types=[
        pltpu.VMEM((_R * _HD,), F32),
        pltpu.VMEM((_BC,), I32),
        pltpu.VMEM((_BC,), I32),
        pltpu.VMEM((_BC,), I32),
        pltpu.VMEM((_BC, _HD), F32),
        pltpu.VMEM((_BC, _HD), F32),
        pltpu.VMEM((_BC,), F32),
        pltpu.SemaphoreType.DMA,
        pltpu.SemaphoreType.DMA,
    ],
    compiler_params=pltpu.CompilerParams(needs_layout_passes=False,
                                         use_tc_tiling_on_sc=False),
)(_score_body)


def kernel(node_emb, edge_index, edge_type, W, b_lin, rel_emb_gat,
           a_src, a_dst, a_rel, bias_out, rel_emb_scorer,
           src_ids, rel_ids, dst_ids):
    haug, adstt, relsc = _prep(node_emb, W, b_lin.reshape(1, _HD),
                               a_src, a_dst, a_rel, rel_emb_gat)
    agg2 = _edge(haug, edge_index[0], edge_index[1], edge_type,
                 adstt.reshape(-1), relsc.reshape(-1),
                 rel_emb_gat.reshape(-1))
    refined = _final(agg2, bias_out.reshape(1, _HD))
    return _score(refined, rel_emb_scorer.reshape(-1),
                  src_ids, rel_ids, dst_ids)
